# Initial kernel scaffold; baseline (speedup 1.0000x reference)
import jax
import jax.numpy as jnp
from jax.experimental import pallas as pl


def kernel(x, table):
    raise NotImplementedError("write your pallas kernel here")


# speedup vs baseline: 3.6273x; 3.6273x over previous
import functools

import jax
import jax.numpy as jnp
from jax import lax
from jax.experimental import pallas as pl
from jax.experimental.pallas import tpu as pltpu
from jax.experimental.pallas import tpu_sc as plsc

_BATCH = 4
_SEQ = 8192
_DIM = 1024
_NUM_WORKERS = 32
_ROWS_PER_W = _SEQ // _NUM_WORKERS
_CHUNK = 64


def _sc_broadcast(table):
    mesh = plsc.VectorSubcoreMesh(core_axis_name="c", subcore_axis_name="s")

    @functools.partial(
        pl.kernel,
        mesh=mesh,
        out_type=jax.ShapeDtypeStruct((_BATCH, _SEQ, _DIM), jnp.float32),
        scratch_types=[
            pltpu.VMEM((_CHUNK, _DIM), jnp.float32),
        ],
    )
    def k(table_hbm, out_hbm, buf):
        wid = lax.axis_index("s") * 2 + lax.axis_index("c")
        base = wid * _ROWS_PER_W
        for i in range(_ROWS_PER_W // _CHUNK):
            row = base + i * _CHUNK
            pltpu.sync_copy(table_hbm.at[pl.ds(row, _CHUNK)], buf)
            for b in range(_BATCH):
                pltpu.sync_copy(buf, out_hbm.at[b, pl.ds(row, _CHUNK)])

    return k(table)


def kernel(x, table):
    del x
    return _sc_broadcast(table)
